# baseline (speedup 1.0000x reference)
import functools

import jax
import jax.numpy as jnp
from jax import lax
from jax.experimental import pallas as pl
from jax.experimental.pallas import tpu as pltpu
from jax.experimental.pallas import tpu_sc as plsc

NC = 2
NS = 16
NW = NC * NS
L = 16
CH = 128


def _wid():
    return lax.axis_index("s") * NC + lax.axis_index("c")


def _zero_vmem(ref, rows, cols):
    z = jnp.zeros((L,), jnp.float32)

    def body(i, _):
        for k in range(cols // L):
            ref[i, pl.ds(k * L, L)] = z
        return 0

    lax.fori_loop(0, rows, body, 0)


def _deg_kernel_body(npad, nchunk, col_hbm, deg_out, colv, zbuf, deg_sh):
    cid = lax.axis_index("c")
    sid = lax.axis_index("s")
    wid = _wid()
    rows_per = npad // NS

    pltpu.sync_copy(col_hbm.at[pl.ds(wid * nchunk, nchunk)], colv)

    _zero_vmem(zbuf, CH, L)
    for j in range(rows_per // CH):
        pltpu.sync_copy(zbuf, deg_sh.at[pl.ds(sid * rows_per + j * CH, CH)])
    plsc.subcore_barrier()

    ubuf = zbuf
    one_hot = jnp.where(lax.iota(jnp.int32, L) == 0, 1.0, 0.0).astype(jnp.float32)

    def set_row(i, _):
        ubuf[i, pl.ds(0, L)] = one_hot
        return 0

    lax.fori_loop(0, CH, set_row, 0)

    def edge_chunk(j, _):
        pltpu.sync_copy(ubuf, deg_sh.at[colv.at[j]], add=True)
        return 0

    lax.fori_loop(0, nchunk, edge_chunk, 0)
    plsc.subcore_barrier()

    pltpu.sync_copy(
        deg_sh.at[pl.ds(sid * rows_per, rows_per)],
        deg_out.at[cid, pl.ds(sid * rows_per, rows_per)],
    )


def _scatter_kernel_body(npad, n0, n1, fout,
                         row_hbm, col_hbm, hs_hbm, acc_out,
                         rowv, colv, gbuf, zbuf, acc_sh, gsems, ssems):
    cid = lax.axis_index("c")
    sid = lax.axis_index("s")
    rows_per = npad // NS

    nchunk = jnp.where(cid == 0, n0, n1)
    base = jnp.where(cid == 0, sid * n0, NS * n0 + sid * n1)

    pltpu.sync_copy(row_hbm.at[pl.ds(base, n0)], rowv)
    pltpu.sync_copy(col_hbm.at[pl.ds(base, n0)], colv)

    _zero_vmem(zbuf, CH, fout)
    for j in range(rows_per // CH):
        pltpu.sync_copy(zbuf, acc_sh.at[pl.ds(sid * rows_per + j * CH, CH)])
    plsc.subcore_barrier()

    nbuf = 4

    def gcopy(j, b):
        return pltpu.make_async_copy(hs_hbm.at[rowv.at[j]], gbuf.at[b], gsems[b])

    def swait(j, b):
        pltpu.make_async_copy(gbuf.at[b], acc_sh.at[colv.at[j]], ssems[b]).wait()

    def sstart(j, b):
        pltpu.async_copy(gbuf.at[b], acc_sh.at[colv.at[j]], ssems[b], add=True)

    def step(g, _):
        for b in range(nbuf):
            j = nbuf * g + b

            @pl.when(jnp.logical_and(j >= nbuf, j - nbuf < nchunk))
            def _():
                swait(j - nbuf, b)

            @pl.when(j < nchunk)
            def _():
                gcopy(j, b).start()

            bc = (b + 2) % nbuf

            @pl.when(jnp.logical_and(j >= 2, j - 2 < nchunk))
            def _():
                gcopy(j - 2, bc).wait()
                sstart(j - 2, bc)
        return 0

    lax.fori_loop(0, nchunk // nbuf + 1, step, 0)
    plsc.subcore_barrier()

    pltpu.sync_copy(
        acc_sh.at[pl.ds(sid * rows_per, rows_per)],
        acc_out.at[cid, pl.ds(sid * rows_per, rows_per)],
    )


def _hs_tc_body(x_ref, w_ref, degp_ref, hs_ref):
    deg = degp_ref[0, :, 0] + degp_ref[1, :, 0] + 1.0
    dis = lax.rsqrt(deg)
    h = jnp.dot(x_ref[...], w_ref[...], preferred_element_type=jnp.float32)
    hs_ref[...] = h * dis[:, None]


def _final_tc_body(acc_ref, hs_ref, degp_ref, b_ref, out_ref):
    deg = degp_ref[0, :, 0] + degp_ref[1, :, 0] + 1.0
    dis = lax.rsqrt(deg)
    s = acc_ref[0] + acc_ref[1] + hs_ref[...]
    out_ref[...] = dis[:, None] * s + b_ref[...]


def kernel(x, edge_index, W, b):
    n, f_in = x.shape
    f_out = W.shape[1]
    e = edge_index.shape[1]

    npad = ((n + 1023) // 1024) * 1024
    tot = ((e + NW * 4 * CH - 1) // (NW * 4 * CH)) * NW * 4
    pair = tot // NS
    n1 = max(4, (pair * 9 // 40) // 4 * 4)
    n0 = pair - n1
    epad = tot * CH
    nchunk = tot // NW

    row = edge_index[0]
    col = edge_index[1]
    pad = epad - e + (n0 - n1) * CH
    row2 = jnp.concatenate(
        [row, jnp.zeros((pad,), jnp.int32)]).reshape(-1, CH)
    col2 = jnp.concatenate(
        [col, jnp.full((pad,), npad - 1, jnp.int32)]).reshape(-1, CH)
    x_pad = jnp.pad(x, ((0, npad - n), (0, 0)))

    mesh = plsc.VectorSubcoreMesh(core_axis_name="c", subcore_axis_name="s")
    sc_params = pltpu.CompilerParams(use_tc_tiling_on_sc=False)

    deg_parts = pl.kernel(
        functools.partial(_deg_kernel_body, npad, nchunk),
        out_type=jax.ShapeDtypeStruct((NC, npad, L), jnp.float32),
        mesh=mesh,
        compiler_params=sc_params,
        scratch_types=[
            pltpu.VMEM((nchunk, CH), jnp.int32),
            pltpu.VMEM((CH, L), jnp.float32),
            pltpu.VMEM_SHARED((npad, L), jnp.float32),
        ],
    )(col2)

    blk = 1024
    grid = npad // blk
    hs = pl.pallas_call(
        _hs_tc_body,
        grid=(grid,),
        in_specs=[
            pl.BlockSpec((blk, f_in), lambda i: (i, 0)),
            pl.BlockSpec((f_in, f_out), lambda i: (0, 0)),
            pl.BlockSpec((NC, blk, L), lambda i: (0, i, 0)),
        ],
        out_specs=pl.BlockSpec((blk, f_out), lambda i: (i, 0)),
        out_shape=jax.ShapeDtypeStruct((npad, f_out), jnp.float32),
    )(x_pad, W, deg_parts)

    acc_parts = pl.kernel(
        functools.partial(_scatter_kernel_body, npad, n0, n1, f_out),
        out_type=jax.ShapeDtypeStruct((NC, npad, f_out), jnp.float32),
        mesh=mesh,
        compiler_params=sc_params,
        scratch_types=[
            pltpu.VMEM((n0, CH), jnp.int32),
            pltpu.VMEM((n0, CH), jnp.int32),
            pltpu.VMEM((4, CH, f_out), jnp.float32),
            pltpu.VMEM((CH, f_out), jnp.float32),
            pltpu.VMEM_SHARED((npad, f_out), jnp.float32),
            (pltpu.SemaphoreType.DMA,) * 4,
            (pltpu.SemaphoreType.DMA,) * 4,
        ],
    )(row2, col2, hs)

    out_pad = pl.pallas_call(
        _final_tc_body,
        grid=(grid,),
        in_specs=[
            pl.BlockSpec((NC, blk, f_out), lambda i: (0, i, 0)),
            pl.BlockSpec((blk, f_out), lambda i: (i, 0)),
            pl.BlockSpec((NC, blk, L), lambda i: (0, i, 0)),
            pl.BlockSpec((1, f_out), lambda i: (0, 0)),
        ],
        out_specs=pl.BlockSpec((blk, f_out), lambda i: (i, 0)),
        out_shape=jax.ShapeDtypeStruct((npad, f_out), jnp.float32),
    )(acc_parts, hs, deg_parts, b.reshape(1, f_out))

    return out_pad[:n]


# speedup vs baseline: 2.1062x; 2.1062x over previous
import functools

import jax
import jax.numpy as jnp
from jax import lax
from jax.experimental import pallas as pl
from jax.experimental.pallas import tpu as pltpu
from jax.experimental.pallas import tpu_sc as plsc

NC = 2
NS = 16
NW = NC * NS
L = 16
CH = 128


def _wid():
    return lax.axis_index("s") * NC + lax.axis_index("c")


def _zero_vmem(ref, rows, cols):
    z = jnp.zeros((L,), jnp.float32)

    def body(i, _):
        for k in range(cols // L):
            ref[i, pl.ds(k * L, L)] = z
        return 0

    lax.fori_loop(0, rows, body, 0)


def _deg_kernel_body(npad, nchunk, col_hbm, deg_out, colv, zbuf, deg_sh):
    cid = lax.axis_index("c")
    sid = lax.axis_index("s")
    wid = _wid()
    rows_per = npad // NS

    pltpu.sync_copy(col_hbm.at[pl.ds(wid * nchunk, nchunk)], colv)

    _zero_vmem(zbuf, CH, L)
    for j in range(rows_per // CH):
        pltpu.sync_copy(zbuf, deg_sh.at[pl.ds(sid * rows_per + j * CH, CH)])
    plsc.subcore_barrier()

    ubuf = zbuf
    one_hot = jnp.where(lax.iota(jnp.int32, L) == 0, 1.0, 0.0).astype(jnp.float32)

    def set_row(i, _):
        ubuf[i, pl.ds(0, L)] = one_hot
        return 0

    lax.fori_loop(0, CH, set_row, 0)

    def edge_chunk(j, _):
        pltpu.sync_copy(ubuf, deg_sh.at[colv.at[j]], add=True)
        return 0

    lax.fori_loop(0, nchunk, edge_chunk, 0)
    plsc.subcore_barrier()

    pltpu.sync_copy(
        deg_sh.at[pl.ds(sid * rows_per, rows_per)],
        deg_out.at[cid, pl.ds(sid * rows_per, rows_per)],
    )


def _scatter_kernel_body(npad, nchunk, fout,
                         row_hbm, col_hbm, hs_hbm, acc_out,
                         rowv, colv, gbuf, acc_sh, hs_sh, gsems, ssems):
    cid = lax.axis_index("c")
    sid = lax.axis_index("s")
    wid = _wid()
    rows_per = npad // NS

    pltpu.sync_copy(hs_hbm.at[pl.ds(sid * rows_per, rows_per)],
                    hs_sh.at[pl.ds(sid * rows_per, rows_per)])
    pltpu.sync_copy(row_hbm.at[pl.ds(wid * nchunk, nchunk)], rowv)
    pltpu.sync_copy(col_hbm.at[pl.ds(wid * nchunk, nchunk)], colv)

    _zero_vmem(gbuf.at[0], CH, fout)
    for j in range(rows_per // CH):
        pltpu.sync_copy(gbuf.at[0], acc_sh.at[pl.ds(sid * rows_per + j * CH, CH)])
    plsc.subcore_barrier()

    nbuf = 3

    def gcopy(j, b):
        return pltpu.make_async_copy(hs_sh.at[rowv.at[j]], gbuf.at[b], gsems[b])

    def swait(j, b):
        pltpu.make_async_copy(gbuf.at[b], acc_sh.at[colv.at[j]], ssems[b]).wait()

    def sstart(j, b):
        pltpu.async_copy(gbuf.at[b], acc_sh.at[colv.at[j]], ssems[b], add=True)

    def step(g, _):
        for b in range(nbuf):
            j = nbuf * g + b

            @pl.when(jnp.logical_and(j >= nbuf, j - nbuf < nchunk))
            def _():
                swait(j - nbuf, b)

            @pl.when(j < nchunk)
            def _():
                gcopy(j, b).start()

            bc = (b + 2) % nbuf

            @pl.when(jnp.logical_and(j >= 1, j - 1 < nchunk))
            def _():
                gcopy(j - 1, bc).wait()
                sstart(j - 1, bc)
        return 0

    lax.fori_loop(0, (nchunk + 5) // nbuf, step, 0)
    plsc.subcore_barrier()

    pltpu.sync_copy(
        acc_sh.at[pl.ds(sid * rows_per, rows_per)],
        acc_out.at[cid, pl.ds(sid * rows_per, rows_per)],
    )


def _hs_tc_body(x_ref, w_ref, degp_ref, hs_ref):
    deg = degp_ref[0, :, 0] + degp_ref[1, :, 0] + 1.0
    dis = lax.rsqrt(deg)
    h = jnp.dot(x_ref[...], w_ref[...], preferred_element_type=jnp.float32)
    hs_ref[...] = h * dis[:, None]


def _final_tc_body(acc_ref, hs_ref, degp_ref, b_ref, out_ref):
    deg = degp_ref[0, :, 0] + degp_ref[1, :, 0] + 1.0
    dis = lax.rsqrt(deg)
    s = acc_ref[0] + acc_ref[1] + hs_ref[...]
    out_ref[...] = dis[:, None] * s + b_ref[...]


def kernel(x, edge_index, W, b):
    n, f_in = x.shape
    f_out = W.shape[1]
    e = edge_index.shape[1]

    npad = ((n + 1023) // 1024) * 1024
    tot = ((e + NW * CH - 1) // (NW * CH)) * NW
    epad = tot * CH
    nchunk = tot // NW

    row = edge_index[0]
    col = edge_index[1]
    pad = epad - e
    row2 = jnp.concatenate(
        [row, jnp.zeros((pad,), jnp.int32)]).reshape(-1, CH)
    col2 = jnp.concatenate(
        [col, jnp.full((pad,), npad - 1, jnp.int32)]).reshape(-1, CH)
    x_pad = jnp.pad(x, ((0, npad - n), (0, 0)))

    mesh = plsc.VectorSubcoreMesh(core_axis_name="c", subcore_axis_name="s")
    sc_params = pltpu.CompilerParams(use_tc_tiling_on_sc=False)

    deg_parts = pl.kernel(
        functools.partial(_deg_kernel_body, npad, nchunk),
        out_type=jax.ShapeDtypeStruct((NC, npad, L), jnp.float32),
        mesh=mesh,
        compiler_params=sc_params,
        scratch_types=[
            pltpu.VMEM((nchunk, CH), jnp.int32),
            pltpu.VMEM((CH, L), jnp.float32),
            pltpu.VMEM_SHARED((npad, L), jnp.float32),
        ],
    )(col2)

    blk = 1024
    grid = npad // blk
    hs = pl.pallas_call(
        _hs_tc_body,
        grid=(grid,),
        in_specs=[
            pl.BlockSpec((blk, f_in), lambda i: (i, 0)),
            pl.BlockSpec((f_in, f_out), lambda i: (0, 0)),
            pl.BlockSpec((NC, blk, L), lambda i: (0, i, 0)),
        ],
        out_specs=pl.BlockSpec((blk, f_out), lambda i: (i, 0)),
        out_shape=jax.ShapeDtypeStruct((npad, f_out), jnp.float32),
    )(x_pad, W, deg_parts)

    acc_parts = pl.kernel(
        functools.partial(_scatter_kernel_body, npad, nchunk, f_out),
        out_type=jax.ShapeDtypeStruct((NC, npad, f_out), jnp.float32),
        mesh=mesh,
        compiler_params=sc_params,
        scratch_types=[
            pltpu.VMEM((nchunk, CH), jnp.int32),
            pltpu.VMEM((nchunk, CH), jnp.int32),
            pltpu.VMEM((3, CH, f_out), jnp.float32),
            pltpu.VMEM_SHARED((npad, f_out), jnp.float32),
            pltpu.VMEM_SHARED((npad, f_out), jnp.float32),
            (pltpu.SemaphoreType.DMA,) * 3,
            (pltpu.SemaphoreType.DMA,) * 3,
        ],
    )(row2, col2, hs)

    out_pad = pl.pallas_call(
        _final_tc_body,
        grid=(grid,),
        in_specs=[
            pl.BlockSpec((NC, blk, f_out), lambda i: (0, i, 0)),
            pl.BlockSpec((blk, f_out), lambda i: (i, 0)),
            pl.BlockSpec((NC, blk, L), lambda i: (0, i, 0)),
            pl.BlockSpec((1, f_out), lambda i: (0, 0)),
        ],
        out_specs=pl.BlockSpec((blk, f_out), lambda i: (i, 0)),
        out_shape=jax.ShapeDtypeStruct((npad, f_out), jnp.float32),
    )(acc_parts, hs, deg_parts, b.reshape(1, f_out))

    return out_pad[:n]


# speedup vs baseline: 2.1480x; 1.0199x over previous
import functools

import jax
import jax.numpy as jnp
from jax import lax
from jax.experimental import pallas as pl
from jax.experimental.pallas import tpu as pltpu
from jax.experimental.pallas import tpu_sc as plsc

NC = 2
NS = 16
NW = NC * NS
L = 16
CH = 125


def _wid():
    return lax.axis_index("s") * NC + lax.axis_index("c")


def _zero_vmem(ref, rows, cols):
    z = jnp.zeros((L,), jnp.float32)

    def body(i, _):
        for k in range(cols // L):
            ref[i, pl.ds(k * L, L)] = z
        return 0

    lax.fori_loop(0, rows, body, 0)


def _deg_kernel_body(npad, nchunk, edge_hbm, deg_out, colv, zbuf, deg_sh):
    cid = lax.axis_index("c")
    sid = lax.axis_index("s")
    wid = _wid()
    rows_per = npad // NS

    pltpu.sync_copy(edge_hbm.at[1, pl.ds(wid * nchunk, nchunk)], colv)

    _zero_vmem(zbuf, CH, L)
    for j in range(rows_per // CH):
        pltpu.sync_copy(zbuf, deg_sh.at[pl.ds(sid * rows_per + j * CH, CH)])
    plsc.subcore_barrier()

    ubuf = zbuf
    one_hot = jnp.where(lax.iota(jnp.int32, L) == 0, 1.0, 0.0).astype(jnp.float32)

    def set_row(i, _):
        ubuf[i, pl.ds(0, L)] = one_hot
        return 0

    lax.fori_loop(0, CH, set_row, 0)

    def edge_chunk(j, _):
        pltpu.sync_copy(ubuf, deg_sh.at[colv.at[j]], add=True)
        return 0

    lax.fori_loop(0, nchunk, edge_chunk, 0)
    plsc.subcore_barrier()

    pltpu.sync_copy(
        deg_sh.at[pl.ds(sid * rows_per, rows_per), pl.ds(0, 8)],
        deg_out.at[cid, pl.ds(sid * rows_per, rows_per)],
    )


def _scatter_kernel_body(npad, nchunk, fout,
                         edge_hbm, hs_hbm, acc_out,
                         rowv, colv, gbuf, acc_sh, hs_sh, gsems, ssems):
    cid = lax.axis_index("c")
    sid = lax.axis_index("s")
    wid = _wid()
    rows_per = npad // NS

    pltpu.sync_copy(hs_hbm.at[pl.ds(sid * rows_per, rows_per)],
                    hs_sh.at[pl.ds(sid * rows_per, rows_per)])
    pltpu.sync_copy(edge_hbm.at[0, pl.ds(wid * nchunk, nchunk)], rowv)
    pltpu.sync_copy(edge_hbm.at[1, pl.ds(wid * nchunk, nchunk)], colv)

    _zero_vmem(gbuf.at[0], CH, fout)
    for j in range(rows_per // CH):
        pltpu.sync_copy(gbuf.at[0], acc_sh.at[pl.ds(sid * rows_per + j * CH, CH)])
    plsc.subcore_barrier()

    nbuf = 3

    def gcopy(j, b):
        return pltpu.make_async_copy(hs_sh.at[rowv.at[j]], gbuf.at[b], gsems[b])

    def swait(j, b):
        pltpu.make_async_copy(gbuf.at[b], acc_sh.at[colv.at[j]], ssems[b]).wait()

    def sstart(j, b):
        pltpu.async_copy(gbuf.at[b], acc_sh.at[colv.at[j]], ssems[b], add=True)

    def step(g, _):
        for b in range(nbuf):
            j = nbuf * g + b

            @pl.when(jnp.logical_and(j >= nbuf, j - nbuf < nchunk))
            def _():
                swait(j - nbuf, b)

            @pl.when(j < nchunk)
            def _():
                gcopy(j, b).start()

            bc = (b + 2) % nbuf

            @pl.when(jnp.logical_and(j >= 1, j - 1 < nchunk))
            def _():
                gcopy(j - 1, bc).wait()
                sstart(j - 1, bc)
        return 0

    lax.fori_loop(0, (nchunk + 5) // nbuf, step, 0)
    plsc.subcore_barrier()

    pltpu.sync_copy(
        acc_sh.at[pl.ds(sid * rows_per, rows_per)],
        acc_out.at[cid, pl.ds(sid * rows_per, rows_per)],
    )


def _mm_tc_body(x_ref, w_ref, h_ref):
    h_ref[...] = jnp.dot(x_ref[...], w_ref[...],
                         preferred_element_type=jnp.float32)


def _scale_tc_body(h_ref, degp_ref, hs_ref):
    deg = degp_ref[0, :, 0] + degp_ref[1, :, 0] + 1.0
    dis = lax.rsqrt(deg)
    hs_ref[...] = h_ref[...] * dis[:, None]


def _final_tc_body(acc_ref, hs_ref, degp_ref, b_ref, out_ref):
    deg = degp_ref[0, :, 0] + degp_ref[1, :, 0] + 1.0
    dis = lax.rsqrt(deg)
    s = acc_ref[0] + acc_ref[1] + hs_ref[...]
    out_ref[...] = dis[:, None] * s + b_ref[...]


def kernel(x, edge_index, W, b):
    n, f_in = x.shape
    f_out = W.shape[1]
    e = edge_index.shape[1]

    tot = ((e + NW * CH - 1) // (NW * CH)) * NW
    pad = tot * CH - e
    nchunk = tot // NW
    rows_per = ((n + NS * CH - 1) // (NS * CH)) * CH
    npad = NS * rows_per
    if pad > 0 and npad == n:
        npad += NS * CH
        rows_per += CH

    if pad == 0:
        e3 = jnp.reshape(edge_index, (2, tot, CH))
    else:
        filler = jnp.stack([jnp.zeros((pad,), jnp.int32),
                            jnp.full((pad,), npad - 1, jnp.int32)])
        e3 = jnp.concatenate([edge_index, filler], axis=1).reshape(2, tot, CH)
    x_pad = jnp.pad(x, ((0, npad - n), (0, 0)))

    mesh = plsc.VectorSubcoreMesh(core_axis_name="c", subcore_axis_name="s")
    sc_params = pltpu.CompilerParams(use_tc_tiling_on_sc=False)

    deg_parts = pl.kernel(
        functools.partial(_deg_kernel_body, npad, nchunk),
        out_type=jax.ShapeDtypeStruct((NC, npad, 8), jnp.float32),
        mesh=mesh,
        compiler_params=sc_params,
        scratch_types=[
            pltpu.VMEM((nchunk, CH), jnp.int32),
            pltpu.VMEM((CH, L), jnp.float32),
            pltpu.VMEM_SHARED((npad, L), jnp.float32),
        ],
    )(e3)

    grid = 10
    blk = npad // grid
    h = pl.pallas_call(
        _mm_tc_body,
        grid=(grid,),
        in_specs=[
            pl.BlockSpec((blk, f_in), lambda i: (i, 0)),
            pl.BlockSpec((f_in, f_out), lambda i: (0, 0)),
        ],
        out_specs=pl.BlockSpec((blk, f_out), lambda i: (i, 0)),
        out_shape=jax.ShapeDtypeStruct((npad, f_out), jnp.float32),
    )(x_pad, W)
    hs = pl.pallas_call(
        _scale_tc_body,
        grid=(grid,),
        in_specs=[
            pl.BlockSpec((blk, f_out), lambda i: (i, 0)),
            pl.BlockSpec((NC, blk, 8), lambda i: (0, i, 0)),
        ],
        out_specs=pl.BlockSpec((blk, f_out), lambda i: (i, 0)),
        out_shape=jax.ShapeDtypeStruct((npad, f_out), jnp.float32),
    )(h, deg_parts)

    acc_parts = pl.kernel(
        functools.partial(_scatter_kernel_body, npad, nchunk, f_out),
        out_type=jax.ShapeDtypeStruct((NC, npad, f_out), jnp.float32),
        mesh=mesh,
        compiler_params=sc_params,
        scratch_types=[
            pltpu.VMEM((nchunk, CH), jnp.int32),
            pltpu.VMEM((nchunk, CH), jnp.int32),
            pltpu.VMEM((3, CH, f_out), jnp.float32),
            pltpu.VMEM_SHARED((npad, f_out), jnp.float32),
            pltpu.VMEM_SHARED((npad, f_out), jnp.float32),
            (pltpu.SemaphoreType.DMA,) * 3,
            (pltpu.SemaphoreType.DMA,) * 3,
        ],
    )(e3, hs)

    out_pad = pl.pallas_call(
        _final_tc_body,
        grid=(grid,),
        in_specs=[
            pl.BlockSpec((NC, blk, f_out), lambda i: (0, i, 0)),
            pl.BlockSpec((blk, f_out), lambda i: (i, 0)),
            pl.BlockSpec((NC, blk, 8), lambda i: (0, i, 0)),
            pl.BlockSpec((1, f_out), lambda i: (0, 0)),
        ],
        out_specs=pl.BlockSpec((blk, f_out), lambda i: (i, 0)),
        out_shape=jax.ShapeDtypeStruct((npad, f_out), jnp.float32),
    )(acc_parts, hs, deg_parts, b.reshape(1, f_out))

    return out_pad[:n]
